# baseline (speedup 1.0000x reference)
import jax
import jax.numpy as jnp
from jax.experimental import pallas as pl
from jax.experimental.pallas import tpu as pltpu

NMS_THRESH = 0.5
SUBJECT_CATEGORY_ID = 0
K = 100
NEG = -1e30

_DOT = dict(precision=jax.lax.Precision.HIGHEST, preferred_element_type=jnp.float32)


def _topk_body(obj_ref, vals_ref, idxs_ref, xs_ref):
    B, Q, CL = obj_ref.shape

    x = obj_ref[...]
    m = jnp.max(x, axis=-1, keepdims=True)
    e = jnp.exp(x - m)
    xs_ref[...] = e / jnp.sum(e, axis=-1, keepdims=True)

    flat_iota = (jax.lax.broadcasted_iota(jnp.int32, (Q, CL), 0) * CL
                 + jax.lax.broadcasted_iota(jnp.int32, (Q, CL), 1))[None]
    lane128 = jax.lax.broadcasted_iota(jnp.int32, (B, 128), 1)
    BIG = jnp.int32(2 ** 30)

    def pop_step(t, carry):
        vals, idxs = carry
        xcur = xs_ref[...]
        mv = jnp.max(jnp.max(xcur, axis=2), axis=1)
        m3 = mv[:, None, None]
        cand = jnp.where(xcur == m3, flat_iota, BIG)
        win = jnp.min(jnp.min(cand, axis=2), axis=1)
        xs_ref[...] = jnp.where(flat_iota == win[:, None, None], -1.0, xcur)
        vals = jnp.where(lane128 == t, mv[:, None], vals)
        idxs = jnp.where(lane128 == t, win[:, None], idxs)
        return vals, idxs

    vals, idxs = jax.lax.fori_loop(
        0, K, pop_step,
        (jnp.zeros((B, 128), jnp.float32), jnp.zeros((B, 128), jnp.int32)))
    vals_ref[...] = vals
    idxs_ref[...] = idxs


def _tail_body(verb_ref, box_ref, ts_ref, cmT_ref, vals_ref, idxs_ref,
               hoi_ref, lbl_ref, boxk_ref, keep_ref, sup_ref):
    Q, V = verb_ref.shape[1], verb_ref.shape[2]

    topk_idx = idxs_ref[0, 0, :K]
    obj_scores = vals_ref[0, 0, :K]
    obj_labels = jnp.bitwise_and(topk_idx, 127)
    topq = jnp.right_shift(topk_idx, 7)
    lbl_ref[0, 0, :] = obj_labels

    q_iota = jax.lax.broadcasted_iota(jnp.int32, (K, Q), 1)
    oh = (topq[:, None] == q_iota).astype(jnp.float32)

    verb_sig = 1.0 / (1.0 + jnp.exp(-verb_ref[0]))
    vs = jax.lax.dot_general(oh, verb_sig, (((1,), (0,)), ((), ())), **_DOT)

    b = box_ref[0]
    hgt = ts_ref[0, :, 0:1]
    wdt = ts_ref[0, :, 1:2]
    scale8 = jnp.concatenate([wdt, hgt, wdt, hgt, wdt, hgt, wdt, hgt],
                             axis=-1)
    scx, scy, sw, sh = b[:, 0:1], b[:, 1:2], b[:, 2:3], b[:, 3:4]
    ocx, ocy, ow, oh_ = b[:, 4:5], b[:, 5:6], b[:, 6:7], b[:, 7:8]
    conv = jnp.concatenate([
        scx - 0.5 * sw, scy - 0.5 * sh, scx + 0.5 * sw, scy + 0.5 * sh,
        ocx - 0.5 * ow, ocy - 0.5 * oh_, ocx + 0.5 * ow, ocy + 0.5 * oh_,
    ], axis=-1) * scale8
    gk = jax.lax.dot_general(oh, conv, (((1,), (0,)), ((), ())), **_DOT)
    gt = jax.lax.dot_general(conv, oh, (((0,), (1,)), ((), ())), **_DOT)
    boxk_ref[0] = gk

    c_iota = jax.lax.broadcasted_iota(jnp.int32, (K, 81), 1)
    ohl = (obj_labels[:, None] == c_iota).astype(jnp.float32)
    mask = jax.lax.dot_general(ohl, cmT_ref[...], (((1,), (0,)), ((), ())),
                               **_DOT)
    hoi = (vs * obj_scores[:, None]) * mask
    hoi_ref[0] = hoi

    ms = jnp.max(hoi, axis=-1)
    s_col = ms[:, None]
    s_row = ms[None, :]
    i_col = jax.lax.broadcasted_iota(jnp.int32, (K, K), 0)
    j_row = jax.lax.broadcasted_iota(jnp.int32, (K, K), 1)
    beats = (s_row > s_col) | ((s_row == s_col) & (j_row > i_col))
    rank = jnp.sum(beats.astype(jnp.int32), axis=1)
    p_iota = jax.lax.broadcasted_iota(jnp.int32, (K, K), 0)
    P = (rank[None, :] == p_iota).astype(jnp.float32)

    lblf = obj_labels.astype(jnp.float32)
    feats = jnp.concatenate([gk, lblf[:, None]], axis=-1)
    sfeat = jax.lax.dot_general(P, feats, (((1,), (0,)), ((), ())), **_DOT)
    st = jax.lax.dot_general(gt, P, (((1,), (1,)), ((), ())), **_DOT)
    slbl_row = jax.lax.dot_general(lblf[None, :], P, (((1,), (1,)), ((), ())),
                                   **_DOT)

    def iou_terms(x1c, y1c, x2c, y2c, x1r, y1r, x2r, y2r):
        area_c = (x2c - x1c + 1) * (y2c - y1c + 1)
        area_r = (x2r - x1r + 1) * (y2r - y1r + 1)
        w = jnp.maximum(0.0, jnp.minimum(x2c, x2r) - jnp.maximum(x1c, x1r) + 1)
        h = jnp.maximum(0.0, jnp.minimum(y2c, y2r) - jnp.maximum(y1c, y1r) + 1)
        inter = w * h
        union = area_c + area_r - inter
        return inter, union

    si, su = iou_terms(sfeat[:, 0:1], sfeat[:, 1:2], sfeat[:, 2:3],
                       sfeat[:, 3:4], st[0:1, :], st[1:2, :], st[2:3, :],
                       st[3:4, :])
    oi, ou = iou_terms(sfeat[:, 4:5], sfeat[:, 5:6], sfeat[:, 6:7],
                       sfeat[:, 7:8], st[4:5, :], st[5:6, :], st[6:7, :],
                       st[7:8, :])
    ovr = si / su * (oi / ou)
    same = sfeat[:, 8:9] == slbl_row
    sup_ref[...] = jnp.where(same & (ovr > NMS_THRESH), 1.0, 0.0)

    laneK = jax.lax.broadcasted_iota(jnp.int32, (1, K), 1)

    def nms_step(p, keep):
        row = sup_ref[pl.ds(p, 1), :]
        hit = jnp.max(row * keep, axis=-1, keepdims=True)
        return jnp.where(laneK == p, 1.0 - hit, keep)

    keep_s = jax.lax.fori_loop(0, K, nms_step, jnp.zeros((1, K), jnp.float32))
    keep_ref[0, 0, :] = jax.lax.dot_general(
        keep_s, P, (((1,), (0,)), ((), ())), **_DOT)[0]


@jax.jit
def kernel(pred_obj_logits, pred_verb_logits, pred_sub_boxes, pred_obj_boxes,
           target_sizes, correct_mat):
    B, Q, C = pred_obj_logits.shape
    V = pred_verb_logits.shape[-1]

    obj_pad = jnp.pad(pred_obj_logits, ((0, 0), (0, 0), (0, 128 - C)),
                      constant_values=NEG)

    vals, idxs = pl.pallas_call(
        _topk_body,
        out_shape=(
            jax.ShapeDtypeStruct((B, 128), jnp.float32),
            jax.ShapeDtypeStruct((B, 128), jnp.int32),
        ),
        scratch_shapes=[pltpu.VMEM((B, Q, 128), jnp.float32)],
    )(obj_pad)

    boxes_in = jnp.concatenate([pred_sub_boxes, pred_obj_boxes], axis=-1)
    cm = jnp.concatenate([correct_mat, jnp.ones((V, 1), correct_mat.dtype)],
                         axis=1)
    cmT = cm.T
    tsf = target_sizes.astype(jnp.float32).reshape(B, 1, 2)
    vals3 = vals.reshape(B, 1, 128)
    idxs3 = idxs.reshape(B, 1, 128)

    hoi, lbl3, boxk, keep3 = pl.pallas_call(
        _tail_body,
        grid=(B,),
        in_specs=[
            pl.BlockSpec((1, Q, V), lambda b: (b, 0, 0)),
            pl.BlockSpec((1, Q, 8), lambda b: (b, 0, 0)),
            pl.BlockSpec((1, 1, 2), lambda b: (b, 0, 0)),
            pl.BlockSpec((C, V), lambda b: (0, 0)),
            pl.BlockSpec((1, 1, 128), lambda b: (b, 0, 0)),
            pl.BlockSpec((1, 1, 128), lambda b: (b, 0, 0)),
        ],
        out_specs=[
            pl.BlockSpec((1, K, V), lambda b: (b, 0, 0)),
            pl.BlockSpec((1, 1, K), lambda b: (b, 0, 0)),
            pl.BlockSpec((1, K, 8), lambda b: (b, 0, 0)),
            pl.BlockSpec((1, 1, K), lambda b: (b, 0, 0)),
        ],
        out_shape=(
            jax.ShapeDtypeStruct((B, K, V), jnp.float32),
            jax.ShapeDtypeStruct((B, 1, K), jnp.int32),
            jax.ShapeDtypeStruct((B, K, 8), jnp.float32),
            jax.ShapeDtypeStruct((B, 1, K), jnp.float32),
        ),
        scratch_shapes=[pltpu.VMEM((K, K), jnp.float32)],
    )(pred_verb_logits, boxes_in, tsf, cmT, vals3, idxs3)

    obj_labels = lbl3.reshape(B, K)
    labels = jnp.concatenate(
        [jnp.full_like(obj_labels, SUBJECT_CATEGORY_ID), obj_labels], axis=1)
    boxes = jnp.concatenate([boxk[..., 0:4], boxk[..., 4:8]], axis=1)
    return hoi, labels, boxes, keep3.reshape(B, K)


# speedup vs baseline: 1.4416x; 1.4416x over previous
import jax
import jax.numpy as jnp
from jax.experimental import pallas as pl
from jax.experimental.pallas import tpu as pltpu

NMS_THRESH = 0.5
SUBJECT_CATEGORY_ID = 0
K = 100
NEG = -1e30

_DOT = dict(precision=jax.lax.Precision.HIGHEST, preferred_element_type=jnp.float32)


def _topk_body(obj_ref, vals_ref, idxs_ref, xs_ref):
    B, Q, CL = obj_ref.shape

    x = obj_ref[...]
    m = jnp.max(x, axis=-1, keepdims=True)
    e = jnp.exp(x - m)
    xs_ref[...] = e / jnp.sum(e, axis=-1, keepdims=True)

    flat_iota = (jax.lax.broadcasted_iota(jnp.int32, (Q, CL), 0) * CL
                 + jax.lax.broadcasted_iota(jnp.int32, (Q, CL), 1))[None]
    lane128 = jax.lax.broadcasted_iota(jnp.int32, (B, 128), 1)
    BIG = jnp.int32(2 ** 30)

    def pop_step(t, carry):
        vals, idxs = carry
        xcur = xs_ref[...]
        mv = jnp.max(jnp.max(xcur, axis=2, keepdims=True), axis=1,
                     keepdims=True)
        cand = jnp.where(xcur == mv, flat_iota, BIG)
        win = jnp.min(jnp.min(cand, axis=2, keepdims=True), axis=1,
                      keepdims=True)
        xs_ref[...] = jnp.where(flat_iota == win, -1.0, xcur)
        vals = jnp.where(lane128 == t, mv[:, :, 0], vals)
        idxs = jnp.where(lane128 == t, win[:, :, 0], idxs)
        return vals, idxs

    vals, idxs = jax.lax.fori_loop(
        0, K, pop_step,
        (jnp.zeros((B, 128), jnp.float32), jnp.zeros((B, 128), jnp.int32)))
    vals_ref[...] = vals
    idxs_ref[...] = idxs


def _tail_body(verb_ref, box_ref, ts_ref, cmT_ref, vals_ref, idxs_ref,
               hoi_ref, lbl_ref, boxk_ref, sup_ref, rank_ref):
    Q, V = verb_ref.shape[1], verb_ref.shape[2]

    topk_idx = idxs_ref[0, 0, :K]
    obj_scores = vals_ref[0, 0, :K]
    obj_labels = jnp.bitwise_and(topk_idx, 127)
    topq = jnp.right_shift(topk_idx, 7)
    lbl_ref[0, 0, :] = obj_labels

    q_iota = jax.lax.broadcasted_iota(jnp.int32, (K, Q), 1)
    oh = (topq[:, None] == q_iota).astype(jnp.float32)

    verb_sig = 1.0 / (1.0 + jnp.exp(-verb_ref[0]))
    vs = jax.lax.dot_general(oh, verb_sig, (((1,), (0,)), ((), ())), **_DOT)

    b = box_ref[0]
    hgt = ts_ref[0, :, 0:1]
    wdt = ts_ref[0, :, 1:2]
    scale8 = jnp.concatenate([wdt, hgt, wdt, hgt, wdt, hgt, wdt, hgt],
                             axis=-1)
    scx, scy, sw, sh = b[:, 0:1], b[:, 1:2], b[:, 2:3], b[:, 3:4]
    ocx, ocy, ow, oh_ = b[:, 4:5], b[:, 5:6], b[:, 6:7], b[:, 7:8]
    conv = jnp.concatenate([
        scx - 0.5 * sw, scy - 0.5 * sh, scx + 0.5 * sw, scy + 0.5 * sh,
        ocx - 0.5 * ow, ocy - 0.5 * oh_, ocx + 0.5 * ow, ocy + 0.5 * oh_,
    ], axis=-1) * scale8
    gk = jax.lax.dot_general(oh, conv, (((1,), (0,)), ((), ())), **_DOT)
    gt = jax.lax.dot_general(conv, oh, (((0,), (1,)), ((), ())), **_DOT)
    boxk_ref[0] = gk

    c_iota = jax.lax.broadcasted_iota(jnp.int32, (K, 81), 1)
    ohl = (obj_labels[:, None] == c_iota).astype(jnp.float32)
    mask = jax.lax.dot_general(ohl, cmT_ref[...], (((1,), (0,)), ((), ())),
                               **_DOT)
    hoi = (vs * obj_scores[:, None]) * mask
    hoi_ref[0] = hoi

    ms = jnp.max(hoi, axis=-1)
    s_col = ms[:, None]
    s_row = ms[None, :]
    i_col = jax.lax.broadcasted_iota(jnp.int32, (K, K), 0)
    j_row = jax.lax.broadcasted_iota(jnp.int32, (K, K), 1)
    beats = (s_row > s_col) | ((s_row == s_col) & (j_row > i_col))
    rank = jnp.sum(beats.astype(jnp.int32), axis=1)
    rank_ref[0, 0, :] = rank
    p_iota = jax.lax.broadcasted_iota(jnp.int32, (K, K), 0)
    P = (rank[None, :] == p_iota).astype(jnp.float32)

    lblf = obj_labels.astype(jnp.float32)
    feats = jnp.concatenate([gk, lblf[:, None]], axis=-1)
    sfeat = jax.lax.dot_general(P, feats, (((1,), (0,)), ((), ())), **_DOT)
    st = jax.lax.dot_general(gt, P, (((1,), (1,)), ((), ())), **_DOT)
    slbl_row = jax.lax.dot_general(lblf[None, :], P, (((1,), (1,)), ((), ())),
                                   **_DOT)

    def iou_terms(x1c, y1c, x2c, y2c, x1r, y1r, x2r, y2r):
        area_c = (x2c - x1c + 1) * (y2c - y1c + 1)
        area_r = (x2r - x1r + 1) * (y2r - y1r + 1)
        w = jnp.maximum(0.0, jnp.minimum(x2c, x2r) - jnp.maximum(x1c, x1r) + 1)
        h = jnp.maximum(0.0, jnp.minimum(y2c, y2r) - jnp.maximum(y1c, y1r) + 1)
        inter = w * h
        union = area_c + area_r - inter
        return inter, union

    si, su = iou_terms(sfeat[:, 0:1], sfeat[:, 1:2], sfeat[:, 2:3],
                       sfeat[:, 3:4], st[0:1, :], st[1:2, :], st[2:3, :],
                       st[3:4, :])
    oi, ou = iou_terms(sfeat[:, 4:5], sfeat[:, 5:6], sfeat[:, 6:7],
                       sfeat[:, 7:8], st[4:5, :], st[5:6, :], st[6:7, :],
                       st[7:8, :])
    ovr = si / su * (oi / ou)
    same = sfeat[:, 8:9] == slbl_row
    sup_ref[0] = jnp.where(same & (ovr > NMS_THRESH), 1.0, 0.0)


def _nms_body(sup_ref, rank_ref, keep_ref):
    B = sup_ref.shape[0]
    rank = rank_ref[:, 0, :]
    laneK = jax.lax.broadcasted_iota(jnp.int32, (B, K), 1)

    def nms_step(p, carry):
        keep_s, keep_o = carry
        row = sup_ref[:, pl.ds(p, 1), :]
        hit = jnp.max(row[:, 0, :] * keep_s, axis=-1, keepdims=True)
        dec = 1.0 - hit
        keep_s = jnp.where(laneK == p, dec, keep_s)
        keep_o = jnp.where(rank == p, dec, keep_o)
        return keep_s, keep_o

    zero = jnp.zeros((B, K), jnp.float32)
    _, keep_o = jax.lax.fori_loop(0, K, nms_step, (zero, zero))
    keep_ref[...] = keep_o


@jax.jit
def kernel(pred_obj_logits, pred_verb_logits, pred_sub_boxes, pred_obj_boxes,
           target_sizes, correct_mat):
    B, Q, C = pred_obj_logits.shape
    V = pred_verb_logits.shape[-1]

    obj_pad = jnp.pad(pred_obj_logits, ((0, 0), (0, 0), (0, 128 - C)),
                      constant_values=NEG)

    vals, idxs = pl.pallas_call(
        _topk_body,
        out_shape=(
            jax.ShapeDtypeStruct((B, 128), jnp.float32),
            jax.ShapeDtypeStruct((B, 128), jnp.int32),
        ),
        scratch_shapes=[pltpu.VMEM((B, Q, 128), jnp.float32)],
    )(obj_pad)

    boxes_in = jnp.concatenate([pred_sub_boxes, pred_obj_boxes], axis=-1)
    cm = jnp.concatenate([correct_mat, jnp.ones((V, 1), correct_mat.dtype)],
                         axis=1)
    cmT = cm.T
    tsf = target_sizes.astype(jnp.float32).reshape(B, 1, 2)
    vals3 = vals.reshape(B, 1, 128)
    idxs3 = idxs.reshape(B, 1, 128)

    hoi, lbl3, boxk, sup, rank3 = pl.pallas_call(
        _tail_body,
        grid=(B,),
        in_specs=[
            pl.BlockSpec((1, Q, V), lambda b: (b, 0, 0)),
            pl.BlockSpec((1, Q, 8), lambda b: (b, 0, 0)),
            pl.BlockSpec((1, 1, 2), lambda b: (b, 0, 0)),
            pl.BlockSpec((C, V), lambda b: (0, 0)),
            pl.BlockSpec((1, 1, 128), lambda b: (b, 0, 0)),
            pl.BlockSpec((1, 1, 128), lambda b: (b, 0, 0)),
        ],
        out_specs=[
            pl.BlockSpec((1, K, V), lambda b: (b, 0, 0)),
            pl.BlockSpec((1, 1, K), lambda b: (b, 0, 0)),
            pl.BlockSpec((1, K, 8), lambda b: (b, 0, 0)),
            pl.BlockSpec((1, K, K), lambda b: (b, 0, 0)),
            pl.BlockSpec((1, 1, K), lambda b: (b, 0, 0)),
        ],
        out_shape=(
            jax.ShapeDtypeStruct((B, K, V), jnp.float32),
            jax.ShapeDtypeStruct((B, 1, K), jnp.int32),
            jax.ShapeDtypeStruct((B, K, 8), jnp.float32),
            jax.ShapeDtypeStruct((B, K, K), jnp.float32),
            jax.ShapeDtypeStruct((B, 1, K), jnp.int32),
        ),
    )(pred_verb_logits, boxes_in, tsf, cmT, vals3, idxs3)

    keep = pl.pallas_call(
        _nms_body,
        out_shape=jax.ShapeDtypeStruct((B, K), jnp.float32),
    )(sup, rank3)

    obj_labels = lbl3.reshape(B, K)
    labels = jnp.concatenate(
        [jnp.full_like(obj_labels, SUBJECT_CATEGORY_ID), obj_labels], axis=1)
    boxes = jnp.concatenate([boxk[..., 0:4], boxk[..., 4:8]], axis=1)
    return hoi, labels, boxes, keep


# speedup vs baseline: 3.3145x; 2.2992x over previous
import jax
import jax.numpy as jnp
from jax.experimental import pallas as pl
from jax.experimental.pallas import tpu as pltpu

NMS_THRESH = 0.5
SUBJECT_CATEGORY_ID = 0
K = 100
NEG = -1e30

_DOT = dict(precision=jax.lax.Precision.HIGHEST, preferred_element_type=jnp.float32)


M_DEPTH = 8


def _topk_body(obj_ref, vals_ref, idxs_ref, xs_ref, rv_ref, ra_ref,
               hs_ref, hc_ref, ptr_ref):
    B, CS, Q = obj_ref.shape

    x = obj_ref[...]
    m = jnp.max(x, axis=1, keepdims=True)
    e = jnp.exp(x - m)
    probs = e / jnp.sum(e, axis=1, keepdims=True)

    ci = jax.lax.broadcasted_iota(jnp.int32, (B, CS, Q), 1)
    BIG = jnp.int32(2 ** 30)

    for j in range(M_DEPTH):
        rm = jnp.max(probs, axis=1, keepdims=True)
        rc = jnp.min(jnp.where(probs == rm, ci, BIG), axis=1,
                     keepdims=True)
        rv_ref[:, j:j + 1, :] = rm
        ra_ref[:, j:j + 1, :] = rc
        probs = jnp.where(ci == rc, -1.0, probs)
    xs_ref[...] = probs

    hs_ref[...] = rv_ref[:, 0, :]
    hc_ref[...] = ra_ref[:, 0, :]
    ptr_ref[...] = jnp.zeros((B, Q), jnp.int32)

    q_iota = jax.lax.broadcasted_iota(jnp.int32, (B, Q), 1)
    lane128 = jax.lax.broadcasted_iota(jnp.int32, (B, 128), 1)

    def pop_step(t, dummy):
        hs = hs_ref[...]
        mv = jnp.max(hs, axis=1, keepdims=True)
        qwin = jnp.min(jnp.where(hs == mv, q_iota, BIG), axis=1,
                       keepdims=True)
        ohq = q_iota == qwin
        cwin = jnp.sum(jnp.where(ohq, hc_ref[...], 0), axis=1,
                       keepdims=True)
        vals_ref[...] = jnp.where(lane128 == t, mv, vals_ref[...])
        idxs_ref[...] = jnp.where(lane128 == t, qwin * 128 + cwin,
                                  idxs_ref[...])

        newptr = ptr_ref[...] + jnp.where(ohq, 1, 0)
        ptr_ref[...] = newptr
        d = jnp.sum(jnp.where(ohq, newptr, 0), axis=1, keepdims=True)
        nv = jnp.zeros((B, 1), jnp.float32)
        nc = jnp.zeros((B, 1), jnp.int32)
        for j in range(1, M_DEPTH):
            sel = d == j
            rvw = jnp.sum(jnp.where(ohq, rv_ref[:, j, :], 0.0), axis=1,
                          keepdims=True)
            raw = jnp.sum(jnp.where(ohq, ra_ref[:, j, :], 0), axis=1,
                          keepdims=True)
            nv = jnp.where(sel, rvw, nv)
            nc = jnp.where(sel, raw, nc)
        hs_ref[...] = jnp.where(ohq, nv, hs_ref[...])
        hc_ref[...] = jnp.where(ohq, nc, hc_ref[...])

        need = d >= M_DEPTH
        pred = jnp.sum(jnp.where(need, 1, 0)) > 0

        @pl.when(pred)
        def _fallback():
            xcur = xs_ref[...]
            ohq3 = ohq[:, None, :]
            rowvals = jnp.max(jnp.where(ohq3, xcur, -1.0), axis=2,
                              keepdims=True)
            fv = jnp.max(rowvals, axis=1, keepdims=True)
            fc = jnp.min(jnp.where(rowvals == fv,
                                   jax.lax.broadcasted_iota(
                                       jnp.int32, (B, CS, 1), 1), BIG),
                         axis=1, keepdims=True)
            kill = ohq3 & (ci == fc) & need[:, :, None]
            xs_ref[...] = jnp.where(kill, -1.0, xcur)
            upd = ohq & need
            hs_ref[...] = jnp.where(upd, fv[:, :, 0], hs_ref[...])
            hc_ref[...] = jnp.where(upd, fc[:, :, 0], hc_ref[...])

        return dummy

    jax.lax.fori_loop(0, K, pop_step, 0)


def _tail_body(verb_ref, box_ref, ts_ref, cmT_ref, vals_ref, idxs_ref,
               hoi_ref, lbl_ref, boxk_ref, sup_ref, rank_ref):
    Q, V = verb_ref.shape[1], verb_ref.shape[2]

    topk_idx = idxs_ref[0, 0, :K]
    obj_scores = vals_ref[0, 0, :K]
    obj_labels = jnp.bitwise_and(topk_idx, 127)
    topq = jnp.right_shift(topk_idx, 7)
    lbl_ref[0, 0, :] = obj_labels

    q_iota = jax.lax.broadcasted_iota(jnp.int32, (K, Q), 1)
    oh = (topq[:, None] == q_iota).astype(jnp.float32)

    verb_sig = 1.0 / (1.0 + jnp.exp(-verb_ref[0]))
    vs = jax.lax.dot_general(oh, verb_sig, (((1,), (0,)), ((), ())), **_DOT)

    b = box_ref[0]
    hgt = ts_ref[0, :, 0:1]
    wdt = ts_ref[0, :, 1:2]
    scale8 = jnp.concatenate([wdt, hgt, wdt, hgt, wdt, hgt, wdt, hgt],
                             axis=-1)
    scx, scy, sw, sh = b[:, 0:1], b[:, 1:2], b[:, 2:3], b[:, 3:4]
    ocx, ocy, ow, oh_ = b[:, 4:5], b[:, 5:6], b[:, 6:7], b[:, 7:8]
    conv = jnp.concatenate([
        scx - 0.5 * sw, scy - 0.5 * sh, scx + 0.5 * sw, scy + 0.5 * sh,
        ocx - 0.5 * ow, ocy - 0.5 * oh_, ocx + 0.5 * ow, ocy + 0.5 * oh_,
    ], axis=-1) * scale8
    gk = jax.lax.dot_general(oh, conv, (((1,), (0,)), ((), ())), **_DOT)
    gt = jax.lax.dot_general(conv, oh, (((0,), (1,)), ((), ())), **_DOT)
    boxk_ref[0] = gk

    c_iota = jax.lax.broadcasted_iota(jnp.int32, (K, 81), 1)
    ohl = (obj_labels[:, None] == c_iota).astype(jnp.float32)
    mask = jax.lax.dot_general(ohl, cmT_ref[...], (((1,), (0,)), ((), ())),
                               **_DOT)
    hoi = (vs * obj_scores[:, None]) * mask
    hoi_ref[0] = hoi

    ms = jnp.max(hoi, axis=-1)
    s_col = ms[:, None]
    s_row = ms[None, :]
    i_col = jax.lax.broadcasted_iota(jnp.int32, (K, K), 0)
    j_row = jax.lax.broadcasted_iota(jnp.int32, (K, K), 1)
    beats = (s_row > s_col) | ((s_row == s_col) & (j_row > i_col))
    rank = jnp.sum(beats.astype(jnp.int32), axis=1)
    rank_ref[0, 0, :] = rank
    p_iota = jax.lax.broadcasted_iota(jnp.int32, (K, K), 0)
    P = (rank[None, :] == p_iota).astype(jnp.float32)

    lblf = obj_labels.astype(jnp.float32)
    feats = jnp.concatenate([gk, lblf[:, None]], axis=-1)
    sfeat = jax.lax.dot_general(P, feats, (((1,), (0,)), ((), ())), **_DOT)
    st = jax.lax.dot_general(gt, P, (((1,), (1,)), ((), ())), **_DOT)
    slbl_row = jax.lax.dot_general(lblf[None, :], P, (((1,), (1,)), ((), ())),
                                   **_DOT)

    def iou_terms(x1c, y1c, x2c, y2c, x1r, y1r, x2r, y2r):
        area_c = (x2c - x1c + 1) * (y2c - y1c + 1)
        area_r = (x2r - x1r + 1) * (y2r - y1r + 1)
        w = jnp.maximum(0.0, jnp.minimum(x2c, x2r) - jnp.maximum(x1c, x1r) + 1)
        h = jnp.maximum(0.0, jnp.minimum(y2c, y2r) - jnp.maximum(y1c, y1r) + 1)
        inter = w * h
        union = area_c + area_r - inter
        return inter, union

    si, su = iou_terms(sfeat[:, 0:1], sfeat[:, 1:2], sfeat[:, 2:3],
                       sfeat[:, 3:4], st[0:1, :], st[1:2, :], st[2:3, :],
                       st[3:4, :])
    oi, ou = iou_terms(sfeat[:, 4:5], sfeat[:, 5:6], sfeat[:, 6:7],
                       sfeat[:, 7:8], st[4:5, :], st[5:6, :], st[6:7, :],
                       st[7:8, :])
    ovr = si / su * (oi / ou)
    same = sfeat[:, 8:9] == slbl_row
    sup_ref[0] = jnp.where(same & (ovr > NMS_THRESH), 1.0, 0.0)


def _nms_body(sup_ref, rank_ref, keep_ref):
    B = sup_ref.shape[0]
    rank = rank_ref[:, 0, :]
    laneK = jax.lax.broadcasted_iota(jnp.int32, (B, K), 1)

    def nms_step(p, carry):
        keep_s, keep_o = carry
        row = sup_ref[:, pl.ds(p, 1), :]
        hit = jnp.max(row[:, 0, :] * keep_s, axis=-1, keepdims=True)
        dec = 1.0 - hit
        keep_s = jnp.where(laneK == p, dec, keep_s)
        keep_o = jnp.where(rank == p, dec, keep_o)
        return keep_s, keep_o

    zero = jnp.zeros((B, K), jnp.float32)
    _, keep_o = jax.lax.fori_loop(0, K, nms_step, (zero, zero))
    keep_ref[...] = keep_o


@jax.jit
def kernel(pred_obj_logits, pred_verb_logits, pred_sub_boxes, pred_obj_boxes,
           target_sizes, correct_mat):
    B, Q, C = pred_obj_logits.shape
    V = pred_verb_logits.shape[-1]

    CS = 88
    obj_t = jnp.pad(pred_obj_logits.transpose(0, 2, 1),
                    ((0, 0), (0, CS - C), (0, 0)), constant_values=NEG)

    vals, idxs = pl.pallas_call(
        _topk_body,
        out_shape=(
            jax.ShapeDtypeStruct((B, 128), jnp.float32),
            jax.ShapeDtypeStruct((B, 128), jnp.int32),
        ),
        scratch_shapes=[
            pltpu.VMEM((B, CS, Q), jnp.float32),
            pltpu.VMEM((B, M_DEPTH, Q), jnp.float32),
            pltpu.VMEM((B, M_DEPTH, Q), jnp.int32),
            pltpu.VMEM((B, Q), jnp.float32),
            pltpu.VMEM((B, Q), jnp.int32),
            pltpu.VMEM((B, Q), jnp.int32),
        ],
    )(obj_t)

    boxes_in = jnp.concatenate([pred_sub_boxes, pred_obj_boxes], axis=-1)
    cm = jnp.concatenate([correct_mat, jnp.ones((V, 1), correct_mat.dtype)],
                         axis=1)
    cmT = cm.T
    tsf = target_sizes.astype(jnp.float32).reshape(B, 1, 2)
    vals3 = vals.reshape(B, 1, 128)
    idxs3 = idxs.reshape(B, 1, 128)

    hoi, lbl3, boxk, sup, rank3 = pl.pallas_call(
        _tail_body,
        grid=(B,),
        in_specs=[
            pl.BlockSpec((1, Q, V), lambda b: (b, 0, 0)),
            pl.BlockSpec((1, Q, 8), lambda b: (b, 0, 0)),
            pl.BlockSpec((1, 1, 2), lambda b: (b, 0, 0)),
            pl.BlockSpec((C, V), lambda b: (0, 0)),
            pl.BlockSpec((1, 1, 128), lambda b: (b, 0, 0)),
            pl.BlockSpec((1, 1, 128), lambda b: (b, 0, 0)),
        ],
        out_specs=[
            pl.BlockSpec((1, K, V), lambda b: (b, 0, 0)),
            pl.BlockSpec((1, 1, K), lambda b: (b, 0, 0)),
            pl.BlockSpec((1, K, 8), lambda b: (b, 0, 0)),
            pl.BlockSpec((1, K, K), lambda b: (b, 0, 0)),
            pl.BlockSpec((1, 1, K), lambda b: (b, 0, 0)),
        ],
        out_shape=(
            jax.ShapeDtypeStruct((B, K, V), jnp.float32),
            jax.ShapeDtypeStruct((B, 1, K), jnp.int32),
            jax.ShapeDtypeStruct((B, K, 8), jnp.float32),
            jax.ShapeDtypeStruct((B, K, K), jnp.float32),
            jax.ShapeDtypeStruct((B, 1, K), jnp.int32),
        ),
    )(pred_verb_logits, boxes_in, tsf, cmT, vals3, idxs3)

    keep = pl.pallas_call(
        _nms_body,
        out_shape=jax.ShapeDtypeStruct((B, K), jnp.float32),
    )(sup, rank3)

    obj_labels = lbl3.reshape(B, K)
    labels = jnp.concatenate(
        [jnp.full_like(obj_labels, SUBJECT_CATEGORY_ID), obj_labels], axis=1)
    boxes = jnp.concatenate([boxk[..., 0:4], boxk[..., 4:8]], axis=1)
    return hoi, labels, boxes, keep


# speedup vs baseline: 5.2682x; 1.5895x over previous
import jax
import jax.numpy as jnp
from jax.experimental import pallas as pl
from jax.experimental.pallas import tpu as pltpu

NMS_THRESH = 0.5
SUBJECT_CATEGORY_ID = 0
K = 100
NEG = -1e30

_DOT = dict(precision=jax.lax.Precision.HIGHEST, preferred_element_type=jnp.float32)


M_DEPTH = 8


def _topk_body(obj_ref, vals_ref, idxs_ref, xs_ref, rv_ref, ra_ref,
               hs_ref, hc_ref, ptr_ref):
    B, CS, Q = obj_ref.shape

    x = obj_ref[...]
    m = jnp.max(x, axis=1, keepdims=True)
    e = jnp.exp(x - m)
    probs = e / jnp.sum(e, axis=1, keepdims=True)

    ci = jax.lax.broadcasted_iota(jnp.int32, (B, CS, Q), 1)
    BIG = jnp.int32(2 ** 30)

    for j in range(M_DEPTH):
        rm = jnp.max(probs, axis=1, keepdims=True)
        rc = jnp.min(jnp.where(probs == rm, ci, BIG), axis=1,
                     keepdims=True)
        rv_ref[:, j:j + 1, :] = rm
        ra_ref[:, j:j + 1, :] = rc
        probs = jnp.where(ci == rc, -1.0, probs)
    xs_ref[...] = probs

    hs_ref[...] = rv_ref[:, 0, :]
    hc_ref[...] = ra_ref[:, 0, :]
    ptr_ref[...] = jnp.zeros((B, Q), jnp.int32)

    q_iota = jax.lax.broadcasted_iota(jnp.int32, (B, Q), 1)
    lane128 = jax.lax.broadcasted_iota(jnp.int32, (B, 128), 1)

    def pop_step(t, dummy):
        hs = hs_ref[...]
        mv = jnp.max(hs, axis=1, keepdims=True)
        qwin = jnp.min(jnp.where(hs == mv, q_iota, BIG), axis=1,
                       keepdims=True)
        ohq = q_iota == qwin
        cwin = jnp.sum(jnp.where(ohq, hc_ref[...], 0), axis=1,
                       keepdims=True)
        vals_ref[...] = jnp.where(lane128 == t, mv, vals_ref[...])
        idxs_ref[...] = jnp.where(lane128 == t, qwin * 128 + cwin,
                                  idxs_ref[...])

        newptr = ptr_ref[...] + jnp.where(ohq, 1, 0)
        ptr_ref[...] = newptr
        d = jnp.sum(jnp.where(ohq, newptr, 0), axis=1, keepdims=True)
        nv = jnp.zeros((B, 1), jnp.float32)
        nc = jnp.zeros((B, 1), jnp.int32)
        for j in range(1, M_DEPTH):
            sel = d == j
            rvw = jnp.sum(jnp.where(ohq, rv_ref[:, j, :], 0.0), axis=1,
                          keepdims=True)
            raw = jnp.sum(jnp.where(ohq, ra_ref[:, j, :], 0), axis=1,
                          keepdims=True)
            nv = jnp.where(sel, rvw, nv)
            nc = jnp.where(sel, raw, nc)
        hs_ref[...] = jnp.where(ohq, nv, hs_ref[...])
        hc_ref[...] = jnp.where(ohq, nc, hc_ref[...])

        need = d >= M_DEPTH
        pred = jnp.sum(jnp.where(need, 1, 0)) > 0

        @pl.when(pred)
        def _fallback():
            xcur = xs_ref[...]
            ohq3 = ohq[:, None, :]
            rowvals = jnp.max(jnp.where(ohq3, xcur, -1.0), axis=2,
                              keepdims=True)
            fv = jnp.max(rowvals, axis=1, keepdims=True)
            fc = jnp.min(jnp.where(rowvals == fv,
                                   jax.lax.broadcasted_iota(
                                       jnp.int32, (B, CS, 1), 1), BIG),
                         axis=1, keepdims=True)
            kill = ohq3 & (ci == fc) & need[:, :, None]
            xs_ref[...] = jnp.where(kill, -1.0, xcur)
            upd = ohq & need
            hs_ref[...] = jnp.where(upd, fv[:, :, 0], hs_ref[...])
            hc_ref[...] = jnp.where(upd, fc[:, :, 0], hc_ref[...])

        return dummy

    jax.lax.fori_loop(0, K, pop_step, 0)


def _tail_body(verb_ref, box_ref, ts_ref, cmT_ref, vals_ref, idxs_ref,
               hoi_ref, lbl_ref, boxk_ref, sup_ref, rank_ref):
    Q, V = verb_ref.shape[1], verb_ref.shape[2]

    topk_idx = idxs_ref[0, :, :K]
    obj_scores = vals_ref[0, :, :K]
    obj_labels = jnp.bitwise_and(topk_idx, 127)
    topq = jnp.right_shift(topk_idx, 7)
    lbl_ref[0] = obj_labels

    kk_i = jax.lax.broadcasted_iota(jnp.int32, (K, K), 0)
    kk_j = jax.lax.broadcasted_iota(jnp.int32, (K, K), 1)
    eye = (kk_i == kk_j).astype(jnp.float32)

    def to_col(row_f32):
        return jax.lax.dot_general(eye, row_f32, (((1,), (1,)), ((), ())),
                                   **_DOT)

    topq_col = to_col(topq.astype(jnp.float32))
    lbl_col = to_col(obj_labels.astype(jnp.float32))
    score_col = to_col(obj_scores)

    q_iota = jax.lax.broadcasted_iota(jnp.int32, (K, Q), 1).astype(jnp.float32)
    oh = (topq_col == q_iota).astype(jnp.float32)

    verb_sig = 1.0 / (1.0 + jnp.exp(-verb_ref[0]))
    vs = jax.lax.dot_general(oh, verb_sig, (((1,), (0,)), ((), ())), **_DOT)

    b = box_ref[0]
    hgt = ts_ref[0, :, 0:1]
    wdt = ts_ref[0, :, 1:2]
    scale8 = jnp.concatenate([wdt, hgt, wdt, hgt, wdt, hgt, wdt, hgt],
                             axis=-1)
    scx, scy, sw, sh = b[:, 0:1], b[:, 1:2], b[:, 2:3], b[:, 3:4]
    ocx, ocy, ow, oh_ = b[:, 4:5], b[:, 5:6], b[:, 6:7], b[:, 7:8]
    conv = jnp.concatenate([
        scx - 0.5 * sw, scy - 0.5 * sh, scx + 0.5 * sw, scy + 0.5 * sh,
        ocx - 0.5 * ow, ocy - 0.5 * oh_, ocx + 0.5 * ow, ocy + 0.5 * oh_,
    ], axis=-1) * scale8
    gk = jax.lax.dot_general(oh, conv, (((1,), (0,)), ((), ())), **_DOT)
    gt = jax.lax.dot_general(conv, oh, (((0,), (1,)), ((), ())), **_DOT)
    boxk_ref[0] = gk

    c_iota = jax.lax.broadcasted_iota(jnp.int32, (K, 81), 1).astype(jnp.float32)
    ohl = (lbl_col == c_iota).astype(jnp.float32)
    mask = jax.lax.dot_general(ohl, cmT_ref[...], (((1,), (0,)), ((), ())),
                               **_DOT)
    hoi = (vs * score_col) * mask
    hoi_ref[0] = hoi

    s_col = jnp.max(hoi, axis=-1, keepdims=True)
    s_row = jax.lax.dot_general(s_col, eye, (((0,), (0,)), ((), ())),
                                **_DOT)
    beats = (s_row > s_col) | ((s_row == s_col) & (kk_j > kk_i))
    rank_col = jnp.sum(beats.astype(jnp.float32), axis=1,
                       keepdims=True)
    rank_row = jax.lax.dot_general(rank_col, eye, (((0,), (0,)), ((), ())),
                                   **_DOT)
    rank_ref[0] = rank_row.astype(jnp.int32)
    P = (rank_row == kk_i.astype(jnp.float32)).astype(jnp.float32)

    feats = jnp.concatenate([gk, lbl_col], axis=-1)
    sfeat = jax.lax.dot_general(P, feats, (((1,), (0,)), ((), ())), **_DOT)
    st = jax.lax.dot_general(gt, P, (((1,), (1,)), ((), ())), **_DOT)
    slbl_row = jax.lax.dot_general(obj_labels.astype(jnp.float32), P,
                                   (((1,), (1,)), ((), ())), **_DOT)

    def iou_terms(x1c, y1c, x2c, y2c, x1r, y1r, x2r, y2r):
        area_c = (x2c - x1c + 1) * (y2c - y1c + 1)
        area_r = (x2r - x1r + 1) * (y2r - y1r + 1)
        w = jnp.maximum(0.0, jnp.minimum(x2c, x2r) - jnp.maximum(x1c, x1r) + 1)
        h = jnp.maximum(0.0, jnp.minimum(y2c, y2r) - jnp.maximum(y1c, y1r) + 1)
        inter = w * h
        union = area_c + area_r - inter
        return inter, union

    si, su = iou_terms(sfeat[:, 0:1], sfeat[:, 1:2], sfeat[:, 2:3],
                       sfeat[:, 3:4], st[0:1, :], st[1:2, :], st[2:3, :],
                       st[3:4, :])
    oi, ou = iou_terms(sfeat[:, 4:5], sfeat[:, 5:6], sfeat[:, 6:7],
                       sfeat[:, 7:8], st[4:5, :], st[5:6, :], st[6:7, :],
                       st[7:8, :])
    ovr = si / su * (oi / ou)
    same = sfeat[:, 8:9] == slbl_row
    sup_ref[0] = jnp.where(same & (ovr > NMS_THRESH), 1.0, 0.0)


def _nms_body(sup_ref, rank_ref, keep_ref):
    B = sup_ref.shape[0]
    rank = rank_ref[:, 0, :]
    laneK = jax.lax.broadcasted_iota(jnp.int32, (B, K), 1)

    def nms_step(p, carry):
        keep_s, keep_o = carry
        row = sup_ref[:, pl.ds(p, 1), :]
        hit = jnp.max(row[:, 0, :] * keep_s, axis=-1, keepdims=True)
        dec = 1.0 - hit
        keep_s = jnp.where(laneK == p, dec, keep_s)
        keep_o = jnp.where(rank == p, dec, keep_o)
        return keep_s, keep_o

    zero = jnp.zeros((B, K), jnp.float32)
    _, keep_o = jax.lax.fori_loop(0, K, nms_step, (zero, zero))
    keep_ref[...] = keep_o


@jax.jit
def kernel(pred_obj_logits, pred_verb_logits, pred_sub_boxes, pred_obj_boxes,
           target_sizes, correct_mat):
    B, Q, C = pred_obj_logits.shape
    V = pred_verb_logits.shape[-1]

    CS = 88
    obj_t = jnp.pad(pred_obj_logits.transpose(0, 2, 1),
                    ((0, 0), (0, CS - C), (0, 0)), constant_values=NEG)

    vals, idxs = pl.pallas_call(
        _topk_body,
        out_shape=(
            jax.ShapeDtypeStruct((B, 128), jnp.float32),
            jax.ShapeDtypeStruct((B, 128), jnp.int32),
        ),
        scratch_shapes=[
            pltpu.VMEM((B, CS, Q), jnp.float32),
            pltpu.VMEM((B, M_DEPTH, Q), jnp.float32),
            pltpu.VMEM((B, M_DEPTH, Q), jnp.int32),
            pltpu.VMEM((B, Q), jnp.float32),
            pltpu.VMEM((B, Q), jnp.int32),
            pltpu.VMEM((B, Q), jnp.int32),
        ],
    )(obj_t)

    boxes_in = jnp.concatenate([pred_sub_boxes, pred_obj_boxes], axis=-1)
    cm = jnp.concatenate([correct_mat, jnp.ones((V, 1), correct_mat.dtype)],
                         axis=1)
    cmT = cm.T
    tsf = target_sizes.astype(jnp.float32).reshape(B, 1, 2)
    vals3 = vals.reshape(B, 1, 128)
    idxs3 = idxs.reshape(B, 1, 128)

    hoi, lbl3, boxk, sup, rank3 = pl.pallas_call(
        _tail_body,
        grid=(B,),
        in_specs=[
            pl.BlockSpec((1, Q, V), lambda b: (b, 0, 0)),
            pl.BlockSpec((1, Q, 8), lambda b: (b, 0, 0)),
            pl.BlockSpec((1, 1, 2), lambda b: (b, 0, 0)),
            pl.BlockSpec((C, V), lambda b: (0, 0)),
            pl.BlockSpec((1, 1, 128), lambda b: (b, 0, 0)),
            pl.BlockSpec((1, 1, 128), lambda b: (b, 0, 0)),
        ],
        out_specs=[
            pl.BlockSpec((1, K, V), lambda b: (b, 0, 0)),
            pl.BlockSpec((1, 1, K), lambda b: (b, 0, 0)),
            pl.BlockSpec((1, K, 8), lambda b: (b, 0, 0)),
            pl.BlockSpec((1, K, K), lambda b: (b, 0, 0)),
            pl.BlockSpec((1, 1, K), lambda b: (b, 0, 0)),
        ],
        out_shape=(
            jax.ShapeDtypeStruct((B, K, V), jnp.float32),
            jax.ShapeDtypeStruct((B, 1, K), jnp.int32),
            jax.ShapeDtypeStruct((B, K, 8), jnp.float32),
            jax.ShapeDtypeStruct((B, K, K), jnp.float32),
            jax.ShapeDtypeStruct((B, 1, K), jnp.int32),
        ),
    )(pred_verb_logits, boxes_in, tsf, cmT, vals3, idxs3)

    keep = pl.pallas_call(
        _nms_body,
        out_shape=jax.ShapeDtypeStruct((B, K), jnp.float32),
    )(sup, rank3)

    obj_labels = lbl3.reshape(B, K)
    labels = jnp.concatenate(
        [jnp.full_like(obj_labels, SUBJECT_CATEGORY_ID), obj_labels], axis=1)
    boxes = jnp.concatenate([boxk[..., 0:4], boxk[..., 4:8]], axis=1)
    return hoi, labels, boxes, keep
